# baseline (speedup 1.0000x reference)
import functools

import jax
import jax.numpy as jnp
from jax import lax
from jax.experimental import pallas as pl
from jax.experimental.pallas import tpu as pltpu
from jax.experimental.pallas import tpu_sc as plsc

DIM = 32
S = 4


def _expand4(src, dst, n, N):
  for c in range(n // 16):
    v = src[pl.ds(c * 16, 16)]
    for k in range(S):
      dst[pl.ds(k * n + c * 16, 16)] = v + (k * N)


def _feat_gather(tbl2, idx_v, n, rowsF, col, sem):
  return [pltpu.async_copy(tbl2.at[f].at[idx_v], rowsF.at[f, pl.ds(col, n)],
                           sem)
          for f in range(DIM)]


def _make_index_resolve(B, nw, n_user, n_item, n_ent):
  bp = B // nw

  mesh = plsc.VectorSubcoreMesh(core_axis_name="c", subcore_axis_name="s")

  @functools.partial(
      pl.kernel,
      mesh=mesh,
      compiler_params=pltpu.CompilerParams(use_tc_tiling_on_sc=False),
      out_type=[
          jax.ShapeDtypeStruct((S, B), jnp.int32),
          jax.ShapeDtypeStruct((S, B), jnp.int32),
          jax.ShapeDtypeStruct((S, B), jnp.int32),
          jax.ShapeDtypeStruct((S, S, B), jnp.int32),
          jax.ShapeDtypeStruct((S, B), jnp.int32),
          jax.ShapeDtypeStruct((S, S, B), jnp.int32),
      ],
      scratch_types=[
          pltpu.VMEM((bp,), jnp.int32),
          pltpu.VMEM((bp,), jnp.int32),
          pltpu.VMEM((bp * S,), jnp.int32),
          pltpu.VMEM((bp * S,), jnp.int32),
          pltpu.VMEM((bp * S,), jnp.int32),
          pltpu.VMEM((bp * S,), jnp.int32),
          pltpu.VMEM((bp * S,), jnp.int32),
          pltpu.VMEM((bp * S,), jnp.int32),
          pltpu.VMEM((bp * S,), jnp.int32),
          pltpu.VMEM((bp * S * S,), jnp.int32),
          pltpu.VMEM((bp * S * S,), jnp.int32),
          pltpu.VMEM((bp * S * S,), jnp.int32),
          pltpu.SemaphoreType.DMA,
          pltpu.SemaphoreType.DMA,
          pltpu.SemaphoreType.DMA,
      ],
  )
  def resolve(ui_hbm, ii_hbm, u2i_hbm, i2u_hbm, ae_hbm, ar_hbm,
              vu_out, vi_out, i1_out, i2_out, r0_out, r1_out,
              ui_v, ii_v, expA, expB, expC, val_u2i, val_i2u, idx1_v, r0_v,
              exp512, idx2_v, r1_v, sadj, sval, osem):
    base = (lax.axis_index("s") * 2 + lax.axis_index("c")) * bp

    pltpu.sync_copy(ui_hbm.at[pl.ds(base, bp)], ui_v)
    pltpu.sync_copy(ii_hbm.at[pl.ds(base, bp)], ii_v)

    _expand4(ii_v, expA, bp, n_ent)
    cp_idx1 = pltpu.async_copy(ae_hbm.at[expA], idx1_v, sadj)
    cp_r0 = pltpu.async_copy(ar_hbm.at[expA], r0_v, sadj)
    _expand4(ui_v, expB, bp, n_user)
    cp_vu = pltpu.async_copy(u2i_hbm.at[expB], val_u2i, sval)
    _expand4(ii_v, expC, bp, n_item)
    cp_vi = pltpu.async_copy(i2u_hbm.at[expC], val_i2u, sval)

    cp_idx1.wait()
    cp_r0.wait()
    _expand4(idx1_v, exp512, bp * S, n_ent)
    cp_idx2 = pltpu.async_copy(ae_hbm.at[exp512], idx2_v, sadj)
    cp_r1 = pltpu.async_copy(ar_hbm.at[exp512], r1_v, sadj)
    ocps = [pltpu.async_copy(r0_v.at[pl.ds(s * bp, bp)],
                             r0_out.at[s, pl.ds(base, bp)], osem)
            for s in range(S)]
    ocps += [pltpu.async_copy(idx1_v.at[pl.ds(s * bp, bp)],
                              i1_out.at[s, pl.ds(base, bp)], osem)
             for s in range(S)]
    cp_vu.wait()
    cp_vi.wait()
    ocps += [pltpu.async_copy(val_u2i.at[pl.ds(s * bp, bp)],
                              vu_out.at[s, pl.ds(base, bp)], osem)
             for s in range(S)]
    ocps += [pltpu.async_copy(val_i2u.at[pl.ds(s * bp, bp)],
                              vi_out.at[s, pl.ds(base, bp)], osem)
             for s in range(S)]
    cp_idx2.wait()
    cp_r1.wait()
    ocps += [pltpu.async_copy(r1_v.at[pl.ds((k * S + s) * bp, bp)],
                              r1_out.at[k, s, pl.ds(base, bp)], osem)
             for k in range(S) for s in range(S)]
    ocps += [pltpu.async_copy(idx2_v.at[pl.ds((k * S + s) * bp, bp)],
                              i2_out.at[k, s, pl.ds(base, bp)], osem)
             for k in range(S) for s in range(S)]
    for cp in ocps:
      cp.wait()

  return resolve


def _make_gather(B, nw):
  bp = B // nw

  mesh = plsc.VectorSubcoreMesh(core_axis_name="c", subcore_axis_name="s")

  @functools.partial(
      pl.kernel,
      mesh=mesh,
      compiler_params=pltpu.CompilerParams(use_tc_tiling_on_sc=False),
      out_type=[
          jax.ShapeDtypeStruct((DIM, B), jnp.float32),
          jax.ShapeDtypeStruct((S, DIM, B), jnp.float32),
          jax.ShapeDtypeStruct((S, DIM, B), jnp.float32),
          jax.ShapeDtypeStruct((DIM, B), jnp.float32),
          jax.ShapeDtypeStruct((S, DIM, B), jnp.float32),
          jax.ShapeDtypeStruct((S, S, DIM, B), jnp.float32),
      ],
      scratch_types=[
          pltpu.VMEM((bp,), jnp.int32),
          pltpu.VMEM((bp,), jnp.int32),
          pltpu.VMEM((bp * S,), jnp.int32),
          pltpu.VMEM((bp * S,), jnp.int32),
          pltpu.VMEM((bp * S,), jnp.int32),
          pltpu.VMEM((bp * S * S,), jnp.int32),
          pltpu.VMEM((DIM, bp * 30), jnp.float32),
          pltpu.SemaphoreType.DMA,
          pltpu.SemaphoreType.DMA,
          pltpu.SemaphoreType.DMA,
          pltpu.SemaphoreType.DMA,
      ],
  )
  def gather(ui_hbm, ii_hbm, vu_hbm, vi_hbm, i1_hbm, i2_hbm, ue_hbm, ee_hbm,
             a_out, en_out, u_out, e0_out, e1_out, e2_out,
             ui_v, ii_v, val_u2i, val_i2u, idx1_v, idx2_v, rowsF,
             sf0, sf1, sf2, osem):
    base = (lax.axis_index("s") * 2 + lax.axis_index("c")) * bp
    cA, cE0, cEN, cU, cE1, cE2 = (0, bp, 2 * bp, 6 * bp, 10 * bp, 14 * bp)

    pltpu.sync_copy(ui_hbm.at[pl.ds(base, bp)], ui_v)
    pltpu.sync_copy(ii_hbm.at[pl.ds(base, bp)], ii_v)
    for s in range(S):
      pltpu.sync_copy(vu_hbm.at[s, pl.ds(base, bp)],
                      val_u2i.at[pl.ds(s * bp, bp)])
      pltpu.sync_copy(vi_hbm.at[s, pl.ds(base, bp)],
                      val_i2u.at[pl.ds(s * bp, bp)])
      pltpu.sync_copy(i1_hbm.at[s, pl.ds(base, bp)],
                      idx1_v.at[pl.ds(s * bp, bp)])
    for k in range(S):
      for s in range(S):
        pltpu.sync_copy(i2_hbm.at[k, s, pl.ds(base, bp)],
                        idx2_v.at[pl.ds((k * S + s) * bp, bp)])

    f0 = _feat_gather(ue_hbm, ui_v, bp, rowsF, cA, sf0)
    f0 += _feat_gather(ee_hbm, ii_v, bp, rowsF, cE0, sf0)
    f2 = _feat_gather(ee_hbm, idx2_v, bp * S * S, rowsF, cE2, sf2)
    f1 = _feat_gather(ee_hbm, idx1_v, bp * S, rowsF, cE1, sf1)
    f1 += _feat_gather(ee_hbm, val_u2i, bp * S, rowsF, cEN, sf1)
    f1 += _feat_gather(ue_hbm, val_i2u, bp * S, rowsF, cU, sf1)
    ocps = []

    for cp in f0:
      cp.wait()
    ocps.append(pltpu.async_copy(rowsF.at[:, pl.ds(cA, bp)],
                                 a_out.at[:, pl.ds(base, bp)], osem))
    ocps.append(pltpu.async_copy(rowsF.at[:, pl.ds(cE0, bp)],
                                 e0_out.at[:, pl.ds(base, bp)], osem))
    for cp in f1:
      cp.wait()
    ocps += [pltpu.async_copy(rowsF.at[:, pl.ds(cE1 + s * bp, bp)],
                              e1_out.at[s, :, pl.ds(base, bp)], osem)
             for s in range(S)]
    ocps += [pltpu.async_copy(rowsF.at[:, pl.ds(cEN + s * bp, bp)],
                              en_out.at[s, :, pl.ds(base, bp)], osem)
             for s in range(S)]
    ocps += [pltpu.async_copy(rowsF.at[:, pl.ds(cU + s * bp, bp)],
                              u_out.at[s, :, pl.ds(base, bp)], osem)
             for s in range(S)]
    for cp in f2:
      cp.wait()
    ocps += [pltpu.async_copy(rowsF.at[:, pl.ds(cE2 + (k * S + s) * bp, bp)],
                              e2_out.at[k, s, :, pl.ds(base, bp)], osem)
             for k in range(S) for s in range(S)]
    for cp in ocps:
      cp.wait()

  return gather


def _soft4(logits):
  mx = jnp.maximum(jnp.maximum(logits[0], logits[1]),
                   jnp.maximum(logits[2], logits[3]))
  es = [jnp.exp(l - mx) for l in logits]
  tot = es[0] + es[1] + es[2] + es[3]
  return [e / tot for e in es]


def _dotk(x, y):
  return jnp.sum(x * y, axis=0, keepdims=True)


def _wsum(att, vs):
  return att[0] * vs[0] + att[1] * vs[1] + att[2] * vs[2] + att[3] * vs[3]


def _tc_body(a_ref, en_ref, u_ref, e0_ref, e1_ref, e2_ref, rf_ref, wr_ref,
             wu_ref, wk_ref, bu_ref, bk_ref, out_ref):
  f32 = jnp.float32
  Bb = a_ref.shape[1]
  row = lax.broadcasted_iota(jnp.int32, (DIM, 1), 0)
  m = (row != 0).astype(f32)
  W16 = wr_ref[16 * DIM:17 * DIM, :]
  wu = wu_ref[...]
  wk = wk_ref[...]
  bu = bu_ref[...]
  bk = bk_ref[...]

  u_t = a_ref[...] * m
  n_ts = [jnp.dot(W16, en_ref[s] * m, preferred_element_type=f32)
          for s in range(S)]
  att = _soft4([_dotk(u_t, n) for n in n_ts])
  ngh = _wsum(att, n_ts)
  ue = jnp.tanh(jnp.dot(wu, u_t + ngh, preferred_element_type=f32) + bu) * m

  i_t = e0_ref[...] * m
  ungh = [jnp.dot(W16, u_ref[s] * m, preferred_element_type=f32)
          for s in range(S)]

  e1s = [e1_ref[s] * m for s in range(S)]
  e2s = [e2_ref[k, s] * m for k in range(S) for s in range(S)]
  X = jnp.concatenate(e1s + e2s, axis=1)
  rvec = jnp.concatenate(
      [rf_ref[t].reshape(1, Bb) for t in range(20)], axis=1)
  acc = jnp.zeros_like(X)
  for r in range(16):
    pr = jnp.dot(wr_ref[DIM * r:DIM * (r + 1), :], X, preferred_element_type=f32)
    acc = acc + (rvec == float(r)).astype(f32) * pr
  etw0 = [acc[:, s * Bb:(s + 1) * Bb] for s in range(S)]
  etw1 = [[acc[:, (S + S * k + s) * Bb:(S + S * k + s + 1) * Bb]
           for k in range(S)] for s in range(S)]

  a0 = _soft4([_dotk(i_t, e) for e in etw0])
  comb0 = i_t + _wsum(a0, etw0)
  comb1 = []
  for s in range(S):
    an = _soft4([_dotk(i_t, e) for e in etw1[s]])
    comb1.append(e1s[s] + _wsum(an, etw1[s]))
  C = jnp.concatenate([comb0] + comb1, axis=1)
  H = jax.nn.relu(jnp.dot(wk, C, preferred_element_type=f32) + bk)
  v0 = H[:, :Bb]
  v1 = [H[:, (1 + s) * Bb:(2 + s) * Bb] for s in range(S)]

  X2 = jnp.concatenate([v * m for v in v1], axis=1)
  rvec2 = jnp.concatenate(
      [rf_ref[t].reshape(1, Bb) for t in range(S)], axis=1)
  acc2 = jnp.zeros_like(X2)
  for r in range(16):
    pr = jnp.dot(wr_ref[DIM * r:DIM * (r + 1), :], X2, preferred_element_type=f32)
    acc2 = acc2 + (rvec2 == float(r)).astype(f32) * pr
  etw2 = [acc2[:, s * Bb:(s + 1) * Bb] for s in range(S)]

  au = _soft4([_dotk(i_t, un) for un in ungh])
  user_agg = _wsum(au, ungh)

  a2 = _soft4([_dotk(i_t, e) for e in etw2])
  comb = v0 * m + _wsum(a2, etw2) + user_agg
  ie = jnp.tanh(jnp.dot(wk, comb, preferred_element_type=f32) + bk) * m

  score = jax.nn.sigmoid(_dotk(ue, ie))
  score = jnp.clip(score, 1e-6, 1e6)
  score = jnp.where(jnp.isnan(score), 0.0, score)
  out_ref[...] = score


def _make_compute(B, Bb):
  nb = B // Bb
  full = lambda shape: pl.BlockSpec(shape, lambda i: tuple(0 for _ in shape))
  return pl.pallas_call(
      _tc_body,
      grid=(nb,),
      in_specs=[
          pl.BlockSpec((DIM, Bb), lambda i: (0, i)),
          pl.BlockSpec((S, DIM, Bb), lambda i: (0, 0, i)),
          pl.BlockSpec((S, DIM, Bb), lambda i: (0, 0, i)),
          pl.BlockSpec((DIM, Bb), lambda i: (0, i)),
          pl.BlockSpec((S, DIM, Bb), lambda i: (0, 0, i)),
          pl.BlockSpec((S, S, DIM, Bb), lambda i: (0, 0, 0, i)),
          pl.BlockSpec((24, Bb), lambda i: (0, i)),
          full((17 * DIM, DIM)),
          full((DIM, DIM)),
          full((DIM, DIM)),
          full((DIM, 1)),
          full((DIM, 1)),
      ],
      out_specs=pl.BlockSpec((1, Bb), lambda i: (0, i)),
      out_shape=jax.ShapeDtypeStruct((1, B), jnp.float32),
  )


def kernel(user_index, item_index, adj_u2i, adj_i2u, adj_entity, adj_relation,
           user_emb, entity_emb, W_R, W_user_agg, b_user_agg, W_kg_agg,
           b_kg_agg, c):
  B = user_index.shape[0]

  ui = user_index.astype(jnp.int32)
  ii = item_index.astype(jnp.int32)
  u2i_f = adj_u2i.astype(jnp.int32).T.reshape(-1)
  i2u_f = adj_i2u.astype(jnp.int32).T.reshape(-1)
  ae_f = adj_entity.astype(jnp.int32).T.reshape(-1)
  ar_f = adj_relation.astype(jnp.int32).T.reshape(-1)
  ue_t = user_emb.astype(jnp.float32).T
  ee_t = entity_emb.astype(jnp.float32).T

  info = plsc.get_sparse_core_info()
  nw = info.num_cores * info.num_subcores

  resolve = _make_index_resolve(B, nw, adj_u2i.shape[0], adj_i2u.shape[0],
                                adj_entity.shape[0])
  VU, VI, I1, I2, R0, R1 = resolve(ui, ii, u2i_f, i2u_f, ae_f, ar_f)

  gather = _make_gather(B, nw)
  A, EN, U, E0, E1, E2 = gather(ui, ii, VU, VI, I1, I2, ue_t, ee_t)

  Rf = jnp.concatenate([R0, R1.reshape(S * S, B)], axis=0).astype(jnp.float32)
  Rf = jnp.pad(Rf, ((0, 4), (0, 0)))

  WRf = W_R.astype(jnp.float32)
  WRT = jnp.transpose(WRf, (0, 2, 1)).reshape(17 * DIM, DIM)

  compute = _make_compute(B, 128)
  score = compute(A, EN, U, E0, E1, E2, Rf,
                  WRT,
                  W_user_agg.astype(jnp.float32).T,
                  W_kg_agg.astype(jnp.float32).T,
                  b_user_agg.astype(jnp.float32).reshape(DIM, 1),
                  b_kg_agg.astype(jnp.float32).reshape(DIM, 1))
  return score.reshape(B)


# speedup vs baseline: 1.0860x; 1.0860x over previous
import functools

import jax
import jax.numpy as jnp
from jax import lax
from jax.experimental import pallas as pl
from jax.experimental.pallas import tpu as pltpu
from jax.experimental.pallas import tpu_sc as plsc

DIM = 32
S = 4


def _expand4(src, dst, n, N):
  for c in range(n // 16):
    v = src[pl.ds(c * 16, 16)]
    for k in range(S):
      dst[pl.ds(k * n + c * 16, 16)] = v + (k * N)


def _feat_gather(tbl2, idx_v, n, rowsF, col, sem):
  return [pltpu.async_copy(tbl2.at[f].at[idx_v], rowsF.at[f, pl.ds(col, n)],
                           sem)
          for f in range(DIM)]


def _make_index_resolve(B, nw, n_user, n_item, n_ent):
  bp = B // nw

  mesh = plsc.VectorSubcoreMesh(core_axis_name="c", subcore_axis_name="s")

  @functools.partial(
      pl.kernel,
      mesh=mesh,
      compiler_params=pltpu.CompilerParams(use_tc_tiling_on_sc=False),
      out_type=[
          jax.ShapeDtypeStruct((S, B), jnp.int32),
          jax.ShapeDtypeStruct((S, B), jnp.int32),
          jax.ShapeDtypeStruct((S, B), jnp.int32),
          jax.ShapeDtypeStruct((S, S, B), jnp.int32),
          jax.ShapeDtypeStruct((S, B), jnp.int32),
          jax.ShapeDtypeStruct((S, S, B), jnp.int32),
      ],
      scratch_types=[
          pltpu.VMEM((bp,), jnp.int32),
          pltpu.VMEM((bp,), jnp.int32),
          pltpu.VMEM((bp * S,), jnp.int32),
          pltpu.VMEM((bp * S,), jnp.int32),
          pltpu.VMEM((bp * S,), jnp.int32),
          pltpu.VMEM((bp * S,), jnp.int32),
          pltpu.VMEM((bp * S,), jnp.int32),
          pltpu.VMEM((bp * S,), jnp.int32),
          pltpu.VMEM((bp * S,), jnp.int32),
          pltpu.VMEM((bp * S * S,), jnp.int32),
          pltpu.VMEM((bp * S * S,), jnp.int32),
          pltpu.VMEM((bp * S * S,), jnp.int32),
          pltpu.SemaphoreType.DMA,
          pltpu.SemaphoreType.DMA,
          pltpu.SemaphoreType.DMA,
      ],
  )
  def resolve(ui_hbm, ii_hbm, u2i_hbm, i2u_hbm, ae_hbm, ar_hbm,
              vu_out, vi_out, i1_out, i2_out, r0_out, r1_out,
              ui_v, ii_v, expA, expB, expC, val_u2i, val_i2u, idx1_v, r0_v,
              exp512, idx2_v, r1_v, sadj, sval, osem):
    base = (lax.axis_index("s") * 2 + lax.axis_index("c")) * bp

    pltpu.sync_copy(ui_hbm.at[pl.ds(base, bp)], ui_v)
    pltpu.sync_copy(ii_hbm.at[pl.ds(base, bp)], ii_v)

    _expand4(ii_v, expA, bp, n_ent)
    cp_idx1 = pltpu.async_copy(ae_hbm.at[expA], idx1_v, sadj)
    cp_r0 = pltpu.async_copy(ar_hbm.at[expA], r0_v, sadj)
    _expand4(ui_v, expB, bp, n_user)
    cp_vu = pltpu.async_copy(u2i_hbm.at[expB], val_u2i, sval)
    _expand4(ii_v, expC, bp, n_item)
    cp_vi = pltpu.async_copy(i2u_hbm.at[expC], val_i2u, sval)

    cp_idx1.wait()
    cp_r0.wait()
    _expand4(idx1_v, exp512, bp * S, n_ent)
    cp_idx2 = pltpu.async_copy(ae_hbm.at[exp512], idx2_v, sadj)
    cp_r1 = pltpu.async_copy(ar_hbm.at[exp512], r1_v, sadj)
    ocps = [pltpu.async_copy(r0_v.at[pl.ds(s * bp, bp)],
                             r0_out.at[s, pl.ds(base, bp)], osem)
            for s in range(S)]
    ocps += [pltpu.async_copy(idx1_v.at[pl.ds(s * bp, bp)],
                              i1_out.at[s, pl.ds(base, bp)], osem)
             for s in range(S)]
    cp_vu.wait()
    cp_vi.wait()
    ocps += [pltpu.async_copy(val_u2i.at[pl.ds(s * bp, bp)],
                              vu_out.at[s, pl.ds(base, bp)], osem)
             for s in range(S)]
    ocps += [pltpu.async_copy(val_i2u.at[pl.ds(s * bp, bp)],
                              vi_out.at[s, pl.ds(base, bp)], osem)
             for s in range(S)]
    cp_idx2.wait()
    cp_r1.wait()
    ocps += [pltpu.async_copy(r1_v.at[pl.ds((k * S + s) * bp, bp)],
                              r1_out.at[k, s, pl.ds(base, bp)], osem)
             for k in range(S) for s in range(S)]
    ocps += [pltpu.async_copy(idx2_v.at[pl.ds((k * S + s) * bp, bp)],
                              i2_out.at[k, s, pl.ds(base, bp)], osem)
             for k in range(S) for s in range(S)]
    for cp in ocps:
      cp.wait()

  return resolve


def _make_gather(B, nw):
  bp = B // nw

  mesh = plsc.VectorSubcoreMesh(core_axis_name="c", subcore_axis_name="s")

  @functools.partial(
      pl.kernel,
      mesh=mesh,
      compiler_params=pltpu.CompilerParams(use_tc_tiling_on_sc=False),
      out_type=[
          jax.ShapeDtypeStruct((DIM, B), jnp.float32),
          jax.ShapeDtypeStruct((S, DIM, B), jnp.float32),
          jax.ShapeDtypeStruct((S, DIM, B), jnp.float32),
          jax.ShapeDtypeStruct((DIM, B), jnp.float32),
          jax.ShapeDtypeStruct((S, DIM, B), jnp.float32),
          jax.ShapeDtypeStruct((S, S, DIM, B), jnp.float32),
      ],
      scratch_types=[
          pltpu.VMEM((bp,), jnp.int32),
          pltpu.VMEM((bp,), jnp.int32),
          pltpu.VMEM((bp * S,), jnp.int32),
          pltpu.VMEM((bp * S,), jnp.int32),
          pltpu.VMEM((bp * S,), jnp.int32),
          pltpu.VMEM((bp * S * S,), jnp.int32),
          pltpu.VMEM((DIM, bp * 30), jnp.float32),
          pltpu.SemaphoreType.DMA,
          pltpu.SemaphoreType.DMA,
          pltpu.SemaphoreType.DMA,
          pltpu.SemaphoreType.DMA,
      ],
  )
  def gather(ui_hbm, ii_hbm, vu_hbm, vi_hbm, i1_hbm, i2_hbm, ue_hbm, ee_hbm,
             a_out, en_out, u_out, e0_out, e1_out, e2_out,
             ui_v, ii_v, val_u2i, val_i2u, idx1_v, idx2_v, rowsF,
             sf0, sf1, sf2, osem):
    base = (lax.axis_index("s") * 2 + lax.axis_index("c")) * bp
    cA, cE0, cEN, cU, cE1, cE2 = (0, bp, 2 * bp, 6 * bp, 10 * bp, 14 * bp)

    icps = [pltpu.async_copy(ui_hbm.at[pl.ds(base, bp)], ui_v, osem),
            pltpu.async_copy(ii_hbm.at[pl.ds(base, bp)], ii_v, osem)]
    for s in range(S):
      icps += [
          pltpu.async_copy(vu_hbm.at[s, pl.ds(base, bp)],
                           val_u2i.at[pl.ds(s * bp, bp)], osem),
          pltpu.async_copy(vi_hbm.at[s, pl.ds(base, bp)],
                           val_i2u.at[pl.ds(s * bp, bp)], osem),
          pltpu.async_copy(i1_hbm.at[s, pl.ds(base, bp)],
                           idx1_v.at[pl.ds(s * bp, bp)], osem)]
    for k in range(S):
      for s in range(S):
        icps.append(pltpu.async_copy(i2_hbm.at[k, s, pl.ds(base, bp)],
                                     idx2_v.at[pl.ds((k * S + s) * bp, bp)],
                                     osem))
    for cp in icps:
      cp.wait()

    f0 = _feat_gather(ue_hbm, ui_v, bp, rowsF, cA, sf0)
    f0 += _feat_gather(ee_hbm, ii_v, bp, rowsF, cE0, sf0)
    f2 = _feat_gather(ee_hbm, idx2_v, bp * S * S, rowsF, cE2, sf2)
    f1 = _feat_gather(ee_hbm, idx1_v, bp * S, rowsF, cE1, sf1)
    f1 += _feat_gather(ee_hbm, val_u2i, bp * S, rowsF, cEN, sf1)
    f1 += _feat_gather(ue_hbm, val_i2u, bp * S, rowsF, cU, sf1)
    ocps = []

    for cp in f0:
      cp.wait()
    ocps.append(pltpu.async_copy(rowsF.at[:, pl.ds(cA, bp)],
                                 a_out.at[:, pl.ds(base, bp)], osem))
    ocps.append(pltpu.async_copy(rowsF.at[:, pl.ds(cE0, bp)],
                                 e0_out.at[:, pl.ds(base, bp)], osem))
    for cp in f1:
      cp.wait()
    ocps += [pltpu.async_copy(rowsF.at[:, pl.ds(cE1 + s * bp, bp)],
                              e1_out.at[s, :, pl.ds(base, bp)], osem)
             for s in range(S)]
    ocps += [pltpu.async_copy(rowsF.at[:, pl.ds(cEN + s * bp, bp)],
                              en_out.at[s, :, pl.ds(base, bp)], osem)
             for s in range(S)]
    ocps += [pltpu.async_copy(rowsF.at[:, pl.ds(cU + s * bp, bp)],
                              u_out.at[s, :, pl.ds(base, bp)], osem)
             for s in range(S)]
    for cp in f2:
      cp.wait()
    ocps += [pltpu.async_copy(rowsF.at[:, pl.ds(cE2 + (k * S + s) * bp, bp)],
                              e2_out.at[k, s, :, pl.ds(base, bp)], osem)
             for k in range(S) for s in range(S)]
    for cp in ocps:
      cp.wait()

  return gather


def _soft4(logits):
  mx = jnp.maximum(jnp.maximum(logits[0], logits[1]),
                   jnp.maximum(logits[2], logits[3]))
  es = [jnp.exp(l - mx) for l in logits]
  tot = es[0] + es[1] + es[2] + es[3]
  return [e / tot for e in es]


def _dotk(x, y):
  return jnp.sum(x * y, axis=0, keepdims=True)


def _wsum(att, vs):
  return att[0] * vs[0] + att[1] * vs[1] + att[2] * vs[2] + att[3] * vs[3]


def _tc_body(a_ref, en_ref, u_ref, e0_ref, e1_ref, e2_ref, rf_ref, wr_ref,
             wu_ref, wk_ref, bu_ref, bk_ref, out_ref):
  f32 = jnp.float32
  Bb = a_ref.shape[1]
  row = lax.broadcasted_iota(jnp.int32, (DIM, 1), 0)
  m = (row != 0).astype(f32)
  W16 = wr_ref[16 * DIM:17 * DIM, :]
  wu = wu_ref[...]
  wk = wk_ref[...]
  bu = bu_ref[...]
  bk = bk_ref[...]

  u_t = a_ref[...] * m
  n_ts = [jnp.dot(W16, en_ref[s] * m, preferred_element_type=f32)
          for s in range(S)]
  att = _soft4([_dotk(u_t, n) for n in n_ts])
  ngh = _wsum(att, n_ts)
  ue = jnp.tanh(jnp.dot(wu, u_t + ngh, preferred_element_type=f32) + bu) * m

  i_t = e0_ref[...] * m
  ungh = [jnp.dot(W16, u_ref[s] * m, preferred_element_type=f32)
          for s in range(S)]

  e1s = [e1_ref[s] * m for s in range(S)]
  e2s = [e2_ref[k, s] * m for k in range(S) for s in range(S)]
  X = jnp.concatenate(e1s + e2s, axis=1)
  rvec = jnp.concatenate(
      [rf_ref[t].reshape(1, Bb) for t in range(20)], axis=1)
  acc = jnp.zeros_like(X)
  for r in range(16):
    pr = jnp.dot(wr_ref[DIM * r:DIM * (r + 1), :], X, preferred_element_type=f32)
    acc = acc + (rvec == float(r)).astype(f32) * pr
  etw0 = [acc[:, s * Bb:(s + 1) * Bb] for s in range(S)]
  etw1 = [[acc[:, (S + S * k + s) * Bb:(S + S * k + s + 1) * Bb]
           for k in range(S)] for s in range(S)]

  a0 = _soft4([_dotk(i_t, e) for e in etw0])
  comb0 = i_t + _wsum(a0, etw0)
  comb1 = []
  for s in range(S):
    an = _soft4([_dotk(i_t, e) for e in etw1[s]])
    comb1.append(e1s[s] + _wsum(an, etw1[s]))
  C = jnp.concatenate([comb0] + comb1, axis=1)
  H = jax.nn.relu(jnp.dot(wk, C, preferred_element_type=f32) + bk)
  v0 = H[:, :Bb]
  v1 = [H[:, (1 + s) * Bb:(2 + s) * Bb] for s in range(S)]

  X2 = jnp.concatenate([v * m for v in v1], axis=1)
  rvec2 = jnp.concatenate(
      [rf_ref[t].reshape(1, Bb) for t in range(S)], axis=1)
  acc2 = jnp.zeros_like(X2)
  for r in range(16):
    pr = jnp.dot(wr_ref[DIM * r:DIM * (r + 1), :], X2, preferred_element_type=f32)
    acc2 = acc2 + (rvec2 == float(r)).astype(f32) * pr
  etw2 = [acc2[:, s * Bb:(s + 1) * Bb] for s in range(S)]

  au = _soft4([_dotk(i_t, un) for un in ungh])
  user_agg = _wsum(au, ungh)

  a2 = _soft4([_dotk(i_t, e) for e in etw2])
  comb = v0 * m + _wsum(a2, etw2) + user_agg
  ie = jnp.tanh(jnp.dot(wk, comb, preferred_element_type=f32) + bk) * m

  score = jax.nn.sigmoid(_dotk(ue, ie))
  score = jnp.clip(score, 1e-6, 1e6)
  score = jnp.where(jnp.isnan(score), 0.0, score)
  out_ref[...] = score


def _make_compute(B, Bb):
  nb = B // Bb
  full = lambda shape: pl.BlockSpec(shape, lambda i: tuple(0 for _ in shape))
  return pl.pallas_call(
      _tc_body,
      grid=(nb,),
      in_specs=[
          pl.BlockSpec((DIM, Bb), lambda i: (0, i)),
          pl.BlockSpec((S, DIM, Bb), lambda i: (0, 0, i)),
          pl.BlockSpec((S, DIM, Bb), lambda i: (0, 0, i)),
          pl.BlockSpec((DIM, Bb), lambda i: (0, i)),
          pl.BlockSpec((S, DIM, Bb), lambda i: (0, 0, i)),
          pl.BlockSpec((S, S, DIM, Bb), lambda i: (0, 0, 0, i)),
          pl.BlockSpec((24, Bb), lambda i: (0, i)),
          full((17 * DIM, DIM)),
          full((DIM, DIM)),
          full((DIM, DIM)),
          full((DIM, 1)),
          full((DIM, 1)),
      ],
      out_specs=pl.BlockSpec((1, Bb), lambda i: (0, i)),
      out_shape=jax.ShapeDtypeStruct((1, B), jnp.float32),
  )


def kernel(user_index, item_index, adj_u2i, adj_i2u, adj_entity, adj_relation,
           user_emb, entity_emb, W_R, W_user_agg, b_user_agg, W_kg_agg,
           b_kg_agg, c):
  B = user_index.shape[0]

  ui = user_index.astype(jnp.int32)
  ii = item_index.astype(jnp.int32)
  u2i_f = adj_u2i.astype(jnp.int32).T.reshape(-1)
  i2u_f = adj_i2u.astype(jnp.int32).T.reshape(-1)
  ae_f = adj_entity.astype(jnp.int32).T.reshape(-1)
  ar_f = adj_relation.astype(jnp.int32).T.reshape(-1)
  ue_t = user_emb.astype(jnp.float32).T
  ee_t = entity_emb.astype(jnp.float32).T

  info = plsc.get_sparse_core_info()
  nw = info.num_cores * info.num_subcores

  resolve = _make_index_resolve(B, nw, adj_u2i.shape[0], adj_i2u.shape[0],
                                adj_entity.shape[0])
  VU, VI, I1, I2, R0, R1 = resolve(ui, ii, u2i_f, i2u_f, ae_f, ar_f)

  gather = _make_gather(B, nw)
  A, EN, U, E0, E1, E2 = gather(ui, ii, VU, VI, I1, I2, ue_t, ee_t)

  Rf = jnp.concatenate([R0, R1.reshape(S * S, B)], axis=0).astype(jnp.float32)
  Rf = jnp.pad(Rf, ((0, 4), (0, 0)))

  WRf = W_R.astype(jnp.float32)
  WRT = jnp.transpose(WRf, (0, 2, 1)).reshape(17 * DIM, DIM)

  compute = _make_compute(B, 128)
  score = compute(A, EN, U, E0, E1, E2, Rf,
                  WRT,
                  W_user_agg.astype(jnp.float32).T,
                  W_kg_agg.astype(jnp.float32).T,
                  b_user_agg.astype(jnp.float32).reshape(DIM, 1),
                  b_kg_agg.astype(jnp.float32).reshape(DIM, 1))
  return score.reshape(B)


# speedup vs baseline: 1.0923x; 1.0058x over previous
import functools

import jax
import jax.numpy as jnp
from jax import lax
from jax.experimental import pallas as pl
from jax.experimental.pallas import tpu as pltpu
from jax.experimental.pallas import tpu_sc as plsc

DIM = 32
S = 4


def _expand4(src, dst, n, N):
  for c in range(n // 16):
    v = src[pl.ds(c * 16, 16)]
    for k in range(S):
      dst[pl.ds(k * n + c * 16, 16)] = v + (k * N)


def _feat_gather(tbl2, idx_v, n, rowsF, col, sem):
  return [pltpu.async_copy(tbl2.at[f].at[idx_v], rowsF.at[f, pl.ds(col, n)],
                           sem)
          for f in range(DIM)]


def _make_gather(B, nw, n_user, n_item, n_ent):
  bp = B // nw

  mesh = plsc.VectorSubcoreMesh(core_axis_name="c", subcore_axis_name="s")

  @functools.partial(
      pl.kernel,
      mesh=mesh,
      compiler_params=pltpu.CompilerParams(use_tc_tiling_on_sc=False),
      out_type=[
          jax.ShapeDtypeStruct((DIM, B), jnp.float32),
          jax.ShapeDtypeStruct((S, DIM, B), jnp.float32),
          jax.ShapeDtypeStruct((S, DIM, B), jnp.float32),
          jax.ShapeDtypeStruct((DIM, B), jnp.float32),
          jax.ShapeDtypeStruct((S, DIM, B), jnp.float32),
          jax.ShapeDtypeStruct((S, S, DIM, B), jnp.float32),
          jax.ShapeDtypeStruct((S, B), jnp.int32),
          jax.ShapeDtypeStruct((S, S, B), jnp.int32),
      ],
      scratch_types=[
          pltpu.VMEM((bp,), jnp.int32),
          pltpu.VMEM((bp,), jnp.int32),
          pltpu.VMEM((bp * S,), jnp.int32),
          pltpu.VMEM((bp * S,), jnp.int32),
          pltpu.VMEM((bp * S,), jnp.int32),
          pltpu.VMEM((bp * S,), jnp.int32),
          pltpu.VMEM((bp * S,), jnp.int32),
          pltpu.VMEM((bp * S,), jnp.int32),
          pltpu.VMEM((bp * S,), jnp.int32),
          pltpu.VMEM((bp * S * S,), jnp.int32),
          pltpu.VMEM((bp * S * S,), jnp.int32),
          pltpu.VMEM((bp * S * S,), jnp.int32),
          pltpu.VMEM((DIM, bp * 30), jnp.float32),
          pltpu.SemaphoreType.DMA,
          pltpu.SemaphoreType.DMA,
          pltpu.SemaphoreType.DMA,
          pltpu.SemaphoreType.DMA,
          pltpu.SemaphoreType.DMA,
          pltpu.SemaphoreType.DMA,
      ],
  )
  def gather(ui_hbm, ii_hbm, u2i_hbm, i2u_hbm, ae_hbm, ar_hbm, ue_hbm, ee_hbm,
             a_out, en_out, u_out, e0_out, e1_out, e2_out, r0_out, r1_out,
             ui_v, ii_v, expA, expB, expC, val_u2i, val_i2u, idx1_v, r0_v,
             exp512, idx2_v, r1_v, rowsF, sadj, sval, sf0, sf1, sf2, osem):
    wid = lax.axis_index("s") * 2 + lax.axis_index("c")
    base = wid * bp
    cA, cE0, cEN, cU, cE1, cE2 = (0, bp, 2 * bp, 6 * bp, 10 * bp, 14 * bp)

    pltpu.sync_copy(ui_hbm.at[pl.ds(base, bp)], ui_v)
    pltpu.sync_copy(ii_hbm.at[pl.ds(base, bp)], ii_v)

    _expand4(ii_v, expA, bp, n_ent)
    cp_idx1 = pltpu.async_copy(ae_hbm.at[expA], idx1_v, sadj)
    cp_r0 = pltpu.async_copy(ar_hbm.at[expA], r0_v, sadj)
    _expand4(ui_v, expB, bp, n_user)
    cp_vu = pltpu.async_copy(u2i_hbm.at[expB], val_u2i, sval)
    _expand4(ii_v, expC, bp, n_item)
    cp_vi = pltpu.async_copy(i2u_hbm.at[expC], val_i2u, sval)

    f0 = _feat_gather(ue_hbm, ui_v, bp, rowsF, cA, sf0)
    f0 += _feat_gather(ee_hbm, ii_v, bp, rowsF, cE0, sf0)

    cp_idx1.wait()
    cp_r0.wait()
    _expand4(idx1_v, exp512, bp * S, n_ent)
    cp_idx2 = pltpu.async_copy(ae_hbm.at[exp512], idx2_v, sadj)
    cp_r1 = pltpu.async_copy(ar_hbm.at[exp512], r1_v, sadj)
    f1 = _feat_gather(ee_hbm, idx1_v, bp * S, rowsF, cE1, sf1)
    ocps = [pltpu.async_copy(r0_v.at[pl.ds(k * bp, bp)],
                             r0_out.at[k, pl.ds(base, bp)], osem)
            for k in range(S)]

    cp_vu.wait()
    cp_vi.wait()
    f1 += _feat_gather(ee_hbm, val_u2i, bp * S, rowsF, cEN, sf1)
    f1 += _feat_gather(ue_hbm, val_i2u, bp * S, rowsF, cU, sf1)

    cp_idx2.wait()
    cp_r1.wait()
    f2 = _feat_gather(ee_hbm, idx2_v, bp * S * S, rowsF, cE2, sf2)
    ocps += [pltpu.async_copy(r1_v.at[pl.ds((k * S + s) * bp, bp)],
                              r1_out.at[k, s, pl.ds(base, bp)], osem)
             for k in range(S) for s in range(S)]

    for cp in f0:
      cp.wait()
    ocps.append(pltpu.async_copy(rowsF.at[:, pl.ds(cA, bp)],
                                 a_out.at[:, pl.ds(base, bp)], osem))
    ocps.append(pltpu.async_copy(rowsF.at[:, pl.ds(cE0, bp)],
                                 e0_out.at[:, pl.ds(base, bp)], osem))
    for cp in f1:
      cp.wait()
    ocps += [pltpu.async_copy(rowsF.at[:, pl.ds(cE1 + s * bp, bp)],
                              e1_out.at[s, :, pl.ds(base, bp)], osem)
             for s in range(S)]
    ocps += [pltpu.async_copy(rowsF.at[:, pl.ds(cEN + s * bp, bp)],
                              en_out.at[s, :, pl.ds(base, bp)], osem)
             for s in range(S)]
    ocps += [pltpu.async_copy(rowsF.at[:, pl.ds(cU + s * bp, bp)],
                              u_out.at[s, :, pl.ds(base, bp)], osem)
             for s in range(S)]
    for cp in f2:
      cp.wait()
    ocps += [pltpu.async_copy(rowsF.at[:, pl.ds(cE2 + (k * S + s) * bp, bp)],
                              e2_out.at[k, s, :, pl.ds(base, bp)], osem)
             for k in range(S) for s in range(S)]
    for cp in ocps:
      cp.wait()

  return gather


def _soft4(logits):
  mx = jnp.maximum(jnp.maximum(logits[0], logits[1]),
                   jnp.maximum(logits[2], logits[3]))
  es = [jnp.exp(l - mx) for l in logits]
  tot = es[0] + es[1] + es[2] + es[3]
  return [e / tot for e in es]


def _dotk(x, y):
  return jnp.sum(x * y, axis=0, keepdims=True)


def _wsum(att, vs):
  return att[0] * vs[0] + att[1] * vs[1] + att[2] * vs[2] + att[3] * vs[3]


def _tc_body(a_ref, en_ref, u_ref, e0_ref, e1_ref, e2_ref, rf_ref, wr_ref,
             wu_ref, wk_ref, bu_ref, bk_ref, out_ref):
  f32 = jnp.float32
  Bb = a_ref.shape[1]
  row = lax.broadcasted_iota(jnp.int32, (DIM, 1), 0)
  m = (row != 0).astype(f32)
  W16 = wr_ref[16 * DIM:17 * DIM, :]
  wu = wu_ref[...]
  wk = wk_ref[...]
  bu = bu_ref[...]
  bk = bk_ref[...]

  u_t = a_ref[...] * m
  n_ts = [jnp.dot(W16, en_ref[s] * m, preferred_element_type=f32)
          for s in range(S)]
  att = _soft4([_dotk(u_t, n) for n in n_ts])
  ngh = _wsum(att, n_ts)
  ue = jnp.tanh(jnp.dot(wu, u_t + ngh, preferred_element_type=f32) + bu) * m

  i_t = e0_ref[...] * m
  ungh = [jnp.dot(W16, u_ref[s] * m, preferred_element_type=f32)
          for s in range(S)]

  e1s = [e1_ref[s] * m for s in range(S)]
  e2s = [e2_ref[k, s] * m for k in range(S) for s in range(S)]
  X = jnp.concatenate(e1s + e2s, axis=1)
  rvec = jnp.concatenate(
      [rf_ref[t].reshape(1, Bb) for t in range(20)], axis=1)
  acc = jnp.zeros_like(X)
  for r in range(16):
    pr = jnp.dot(wr_ref[DIM * r:DIM * (r + 1), :], X, preferred_element_type=f32)
    acc = acc + (rvec == float(r)).astype(f32) * pr
  etw0 = [acc[:, s * Bb:(s + 1) * Bb] for s in range(S)]
  etw1 = [[acc[:, (S + S * k + s) * Bb:(S + S * k + s + 1) * Bb]
           for k in range(S)] for s in range(S)]

  a0 = _soft4([_dotk(i_t, e) for e in etw0])
  comb0 = i_t + _wsum(a0, etw0)
  comb1 = []
  for s in range(S):
    an = _soft4([_dotk(i_t, e) for e in etw1[s]])
    comb1.append(e1s[s] + _wsum(an, etw1[s]))
  C = jnp.concatenate([comb0] + comb1, axis=1)
  H = jax.nn.relu(jnp.dot(wk, C, preferred_element_type=f32) + bk)
  v0 = H[:, :Bb]
  v1 = [H[:, (1 + s) * Bb:(2 + s) * Bb] for s in range(S)]

  X2 = jnp.concatenate([v * m for v in v1], axis=1)
  rvec2 = jnp.concatenate(
      [rf_ref[t].reshape(1, Bb) for t in range(S)], axis=1)
  acc2 = jnp.zeros_like(X2)
  for r in range(16):
    pr = jnp.dot(wr_ref[DIM * r:DIM * (r + 1), :], X2, preferred_element_type=f32)
    acc2 = acc2 + (rvec2 == float(r)).astype(f32) * pr
  etw2 = [acc2[:, s * Bb:(s + 1) * Bb] for s in range(S)]

  au = _soft4([_dotk(i_t, un) for un in ungh])
  user_agg = _wsum(au, ungh)

  a2 = _soft4([_dotk(i_t, e) for e in etw2])
  comb = v0 * m + _wsum(a2, etw2) + user_agg
  ie = jnp.tanh(jnp.dot(wk, comb, preferred_element_type=f32) + bk) * m

  score = jax.nn.sigmoid(_dotk(ue, ie))
  score = jnp.clip(score, 1e-6, 1e6)
  score = jnp.where(jnp.isnan(score), 0.0, score)
  out_ref[...] = score


def _make_compute(B, Bb):
  nb = B // Bb
  full = lambda shape: pl.BlockSpec(shape, lambda i: tuple(0 for _ in shape))
  return pl.pallas_call(
      _tc_body,
      grid=(nb,),
      in_specs=[
          pl.BlockSpec((DIM, Bb), lambda i: (0, i)),
          pl.BlockSpec((S, DIM, Bb), lambda i: (0, 0, i)),
          pl.BlockSpec((S, DIM, Bb), lambda i: (0, 0, i)),
          pl.BlockSpec((DIM, Bb), lambda i: (0, i)),
          pl.BlockSpec((S, DIM, Bb), lambda i: (0, 0, i)),
          pl.BlockSpec((S, S, DIM, Bb), lambda i: (0, 0, 0, i)),
          pl.BlockSpec((24, Bb), lambda i: (0, i)),
          full((17 * DIM, DIM)),
          full((DIM, DIM)),
          full((DIM, DIM)),
          full((DIM, 1)),
          full((DIM, 1)),
      ],
      out_specs=pl.BlockSpec((1, Bb), lambda i: (0, i)),
      out_shape=jax.ShapeDtypeStruct((1, B), jnp.float32),
  )


def kernel(user_index, item_index, adj_u2i, adj_i2u, adj_entity, adj_relation,
           user_emb, entity_emb, W_R, W_user_agg, b_user_agg, W_kg_agg,
           b_kg_agg, c):
  B = user_index.shape[0]

  ui = user_index.astype(jnp.int32)
  ii = item_index.astype(jnp.int32)
  u2i_f = adj_u2i.astype(jnp.int32).T.reshape(-1)
  i2u_f = adj_i2u.astype(jnp.int32).T.reshape(-1)
  ae_f = adj_entity.astype(jnp.int32).T.reshape(-1)
  ar_f = adj_relation.astype(jnp.int32).T.reshape(-1)
  ue_t = user_emb.astype(jnp.float32).T
  ee_t = entity_emb.astype(jnp.float32).T

  info = plsc.get_sparse_core_info()
  nw = info.num_cores * info.num_subcores

  gather = _make_gather(B, nw, adj_u2i.shape[0], adj_i2u.shape[0],
                        adj_entity.shape[0])
  A, EN, U, E0, E1, E2, R0, R1 = gather(ui, ii, u2i_f, i2u_f, ae_f, ar_f,
                                        ue_t, ee_t)

  Rf = jnp.concatenate([R0, R1.reshape(S * S, B)], axis=0).astype(jnp.float32)
  Rf = jnp.pad(Rf, ((0, 4), (0, 0)))

  WRf = W_R.astype(jnp.float32)
  WRT = jnp.transpose(WRf, (0, 2, 1)).reshape(17 * DIM, DIM)

  compute = _make_compute(B, 128)
  score = compute(A, EN, U, E0, E1, E2, Rf,
                  WRT,
                  W_user_agg.astype(jnp.float32).T,
                  W_kg_agg.astype(jnp.float32).T,
                  b_user_agg.astype(jnp.float32).reshape(DIM, 1),
                  b_kg_agg.astype(jnp.float32).reshape(DIM, 1))
  return score.reshape(B)
